# baseline (speedup 1.0000x reference)
import functools

import jax
import jax.numpy as jnp
from jax import lax
from jax.experimental import pallas as pl
from jax.experimental.pallas import tpu as pltpu
from jax.experimental.pallas import tpu_sc as plsc

_EPS = 1e-6


def _lane_sum(x, L):
    iota = jnp.arange(L, dtype=jnp.int32)
    dnums = lax.GatherDimensionNumbers(
        offset_dims=(), collapsed_slice_dims=(0,), start_index_map=(0,)
    )
    for k in (1, 2, 4, 8):
        perm = jnp.asarray(iota ^ k, dtype=jnp.int32).reshape(L, 1)
        x = x + lax.gather(
            x,
            perm,
            dimension_numbers=dnums,
            slice_sizes=(1,),
            mode=lax.GatherScatterMode.PROMISE_IN_BOUNDS,
        )
    return x


def _vrsqrt(x):
    i = lax.bitcast_convert_type(x, jnp.int32)
    i = jnp.int32(0x5F3759DF) - lax.shift_right_arithmetic(i, 1)
    y = lax.bitcast_convert_type(i, jnp.float32)
    for _ in range(3):
        y = y * (1.5 - 0.5 * x * y * y)
    return y


@functools.cache
def _make_fused(V, D, B):
    info = plsc.get_sparse_core_info()
    NC, NS, L = info.num_cores, info.num_subcores, info.num_lanes
    NW = NC * NS
    n_vreg = D // L
    rows_per_s = V // NS
    AC = 96
    a_chunks = rows_per_s // AC
    b_per_w = B // NW
    C = 128
    n_chunks = b_per_w // C
    assert V % NS == 0 and rows_per_s % AC == 0 and AC % 8 == 0
    assert B % (8 * NW) == 0 and D % L == 0 and b_per_w % C == 0

    mesh = plsc.VectorSubcoreMesh(core_axis_name="c", subcore_axis_name="s")

    @functools.partial(
        pl.kernel,
        mesh=mesh,
        out_type=(
            jax.ShapeDtypeStruct((B, D), jnp.float32),
            jax.ShapeDtypeStruct((NC * V, D), jnp.float32),
        ),
        scratch_types=[
            pltpu.VMEM((b_per_w,), jnp.int32),
            pltpu.VMEM((C, D), jnp.float32),
            pltpu.VMEM((C, D), jnp.float32),
            pltpu.VMEM((D,), jnp.float32),
            pltpu.SemaphoreType.DMA,
            pltpu.SemaphoreType.DMA,
            pltpu.SemaphoreType.DMA,
            pltpu.SemaphoreType.DMA,
        ],
    )
    def k(ids_hbm, table_hbm, w_hbm, out_hbm, st_hbm,
          idx_v, buf0, buf1, wv, g0, g1, s0, s1):
        core = lax.axis_index("c")
        sid = lax.axis_index("s")
        wid = sid * NC + core
        base = wid * b_per_w
        bufs = (buf0, buf1)
        gsem = (g0, g1)
        ssem = (s0, s1)

        pltpu.sync_copy(w_hbm, wv)
        pltpu.sync_copy(ids_hbm.at[pl.ds(base, b_per_w)], idx_v)
        off = jnp.zeros((L,), jnp.int32) + core * V

        @plsc.parallel_loop(0, b_per_w // L, unroll=4)
        def _(i):
            idx_v[pl.ds(i * L, L)] = idx_v[pl.ds(i * L, L)] + off

        ws = [wv[pl.ds(j * L, L)] for j in range(n_vreg)]

        a_base = sid * rows_per_s
        for ac in range(a_chunks):
            buf = bufs[ac % 2]
            row0 = a_base + ac * AC
            pltpu.sync_copy(table_hbm.at[pl.ds(row0, AC)], buf.at[pl.ds(0, AC)])

            @plsc.parallel_loop(0, AC, unroll=2)
            def _(r):
                a0 = jnp.zeros((L,), jnp.float32)
                a1 = jnp.zeros((L,), jnp.float32)
                a2 = jnp.zeros((L,), jnp.float32)
                for j in range(0, n_vreg, 3):
                    v = buf[r, pl.ds(j * L, L)]
                    a0 = a0 + v * v
                    v = buf[r, pl.ds((j + 1) * L, L)]
                    a1 = a1 + v * v
                    v = buf[r, pl.ds((j + 2) * L, L)]
                    a2 = a2 + v * v
                s = _lane_sum(a0 + a1 + a2, L) * (1.0 / D) + _EPS
                inv = _vrsqrt(s)
                for j in range(n_vreg):
                    buf[r, pl.ds(j * L, L)] = buf[r, pl.ds(j * L, L)] * (ws[j] * inv)

            pltpu.sync_copy(
                buf.at[pl.ds(0, AC)], st_hbm.at[pl.ds(core * V + row0, AC)]
            )

        plsc.subcore_barrier()

        def gather(c):
            return pltpu.async_copy(
                st_hbm.at[idx_v.at[pl.ds(c * C, C)]], bufs[c % 2], gsem[c % 2]
            )

        def store(c):
            return pltpu.async_copy(
                bufs[c % 2], out_hbm.at[pl.ds(base + c * C, C)], ssem[c % 2]
            )

        gathers = [gather(0), gather(1)]
        stores = [None, None]
        for c in range(n_chunks):
            gathers[c % 2].wait()
            stores[c % 2] = store(c)
            if c + 2 < n_chunks:
                stores[c % 2].wait()
                gathers[c % 2] = gather(c + 2)
        stores[(n_chunks - 2) % 2].wait()
        stores[(n_chunks - 1) % 2].wait()

    return k


def kernel(input_ids, tok_embeddings, norm_weight):
    Bt, S = input_ids.shape
    V, D = tok_embeddings.shape
    ids = input_ids.reshape(-1)
    out, _ = _make_fused(V, D, Bt * S)(ids, tok_embeddings, norm_weight)
    return out.reshape(Bt, S, D)


# speedup vs baseline: 1.0342x; 1.0342x over previous
import functools

import jax
import jax.numpy as jnp
from jax import lax
from jax.experimental import pallas as pl
from jax.experimental.pallas import tpu as pltpu
from jax.experimental.pallas import tpu_sc as plsc

_EPS = 1e-6


def _gather16(x, perm, L):
    dnums = lax.GatherDimensionNumbers(
        offset_dims=(), collapsed_slice_dims=(0,), start_index_map=(0,)
    )
    return lax.gather(
        x, perm.reshape(L, 1), dimension_numbers=dnums, slice_sizes=(1,),
        mode=lax.GatherScatterMode.PROMISE_IN_BOUNDS,
    )


def _combine_reduce(vecs, L):
    iota = jnp.arange(L, dtype=jnp.int32)
    k = 1
    while len(vecs) > 1:
        m = jnp.asarray((iota & k) != 0)
        perm = jnp.asarray(iota ^ k, dtype=jnp.int32)
        nxt = []
        for i in range(0, len(vecs), 2):
            x, y = vecs[i], vecs[i + 1]
            nxt.append(
                jnp.where(m, y, x) + _gather16(jnp.where(m, x, y), perm, L)
            )
        vecs = nxt
        k *= 2
    return vecs[0]


def _vrsqrt(x):
    i = lax.bitcast_convert_type(x, jnp.int32)
    i = jnp.int32(0x5F3759DF) - lax.shift_right_arithmetic(i, 1)
    y = lax.bitcast_convert_type(i, jnp.float32)
    for _ in range(3):
        y = y * (1.5 - 0.5 * x * y * y)
    return y


@functools.cache
def _make_fused(V, D, B):
    info = plsc.get_sparse_core_info()
    NC, NS, L = info.num_cores, info.num_subcores, info.num_lanes
    NW = NC * NS
    n_vreg = D // L
    b_per_w = B // NW
    C = 64
    NBUF = 4
    n_chunks = b_per_w // C
    n_rounds = n_chunks // NBUF
    assert B % (8 * NW) == 0 and D % L == 0 and C % L == 0
    assert n_chunks % NBUF == 0 and n_rounds >= 2

    mesh = plsc.VectorSubcoreMesh(core_axis_name="c", subcore_axis_name="s")

    @functools.partial(
        pl.kernel,
        mesh=mesh,
        out_type=jax.ShapeDtypeStruct((B, D), jnp.float32),
        scratch_types=[
            pltpu.VMEM((b_per_w,), jnp.int32),
            pltpu.VMEM((C, D), jnp.float32),
            pltpu.VMEM((C, D), jnp.float32),
            pltpu.VMEM((C, D), jnp.float32),
            pltpu.VMEM((C, D), jnp.float32),
            pltpu.VMEM((D,), jnp.float32),
            pltpu.SemaphoreType.DMA, pltpu.SemaphoreType.DMA,
            pltpu.SemaphoreType.DMA, pltpu.SemaphoreType.DMA,
            pltpu.SemaphoreType.DMA, pltpu.SemaphoreType.DMA,
            pltpu.SemaphoreType.DMA, pltpu.SemaphoreType.DMA,
        ],
    )
    def k(ids_hbm, table_hbm, w_hbm, out_hbm,
          idx_v, buf0, buf1, buf2, buf3, wv,
          g0, g1, g2, g3, s0, s1, s2, s3):
        wid = lax.axis_index("s") * NC + lax.axis_index("c")
        base = wid * b_per_w
        bufs = (buf0, buf1, buf2, buf3)
        gsem = (g0, g1, g2, g3)
        ssem = (s0, s1, s2, s3)

        pltpu.sync_copy(w_hbm, wv)
        pltpu.sync_copy(ids_hbm.at[pl.ds(base, b_per_w)], idx_v)
        ws = [wv[pl.ds(j * L, L)] for j in range(n_vreg)]
        iota = jnp.arange(L, dtype=jnp.int32)

        def gather(c, b):
            return pltpu.async_copy(
                table_hbm.at[idx_v.at[pl.ds(c * C, C)]], bufs[b], gsem[b]
            )

        def wait_gather(b):
            pltpu.make_async_copy(
                table_hbm.at[pl.ds(0, C)], bufs[b], gsem[b]
            ).wait()

        def store(c, b):
            return pltpu.async_copy(
                bufs[b], out_hbm.at[pl.ds(base + c * C, C)], ssem[b]
            )

        def wait_store(b):
            pltpu.make_async_copy(
                bufs[b], out_hbm.at[pl.ds(base, C)], ssem[b]
            ).wait()

        def normalize_chunk(b):
            buf = bufs[b]

            @plsc.parallel_loop(0, C // L)
            def _(g):
                accs = []
                for i in range(L):
                    acc = jnp.zeros((L,), jnp.float32)
                    for j in range(n_vreg):
                        v = buf[g * L + i, pl.ds(j * L, L)]
                        acc = acc + v * v
                    accs.append(acc)
                inv = _vrsqrt(_combine_reduce(accs, L) * (1.0 / D) + _EPS)
                for i in range(L):
                    sb = _gather16(inv, iota * 0 + i, L)
                    for j in range(n_vreg):
                        buf[g * L + i, pl.ds(j * L, L)] = (
                            buf[g * L + i, pl.ds(j * L, L)] * (ws[j] * sb)
                        )

        gather(0, 0)
        gather(1, 1)

        def round_body(t, carry):
            for b in range(NBUF):
                c = t * NBUF + b
                wait_gather(b)
                normalize_chunk(b)
                store(c, b)
                nb = (b + 2) % NBUF
                if b < 2:
                    @pl.when(t >= 1)
                    def _():
                        wait_store(nb)

                    gather(c + 2, nb)
                else:
                    wait_store(nb)

                    @pl.when(t < n_rounds - 1)
                    def _():
                        gather(c + 2, nb)

            return carry

        lax.fori_loop(0, n_rounds, round_body, 0)
        wait_store(2)
        wait_store(3)

    return k


def kernel(input_ids, tok_embeddings, norm_weight):
    Bt, S = input_ids.shape
    V, D = tok_embeddings.shape
    ids = input_ids.reshape(-1)
    out = _make_fused(V, D, Bt * S)(ids, tok_embeddings, norm_weight)
    return out.reshape(Bt, S, D)


# speedup vs baseline: 1.1602x; 1.1219x over previous
import functools

import jax
import jax.numpy as jnp
from jax import lax
from jax.experimental import pallas as pl
from jax.experimental.pallas import tpu as pltpu
from jax.experimental.pallas import tpu_sc as plsc

_EPS = 1e-6


def _gather16(x, perm, L):
    dnums = lax.GatherDimensionNumbers(
        offset_dims=(), collapsed_slice_dims=(0,), start_index_map=(0,)
    )
    return lax.gather(
        x, perm.reshape(L, 1), dimension_numbers=dnums, slice_sizes=(1,),
        mode=lax.GatherScatterMode.PROMISE_IN_BOUNDS,
    )


def _vrsqrt(x):
    i = lax.bitcast_convert_type(x, jnp.int32)
    i = jnp.int32(0x5F3759DF) - lax.shift_right_arithmetic(i, 1)
    y = lax.bitcast_convert_type(i, jnp.float32)
    for _ in range(3):
        y = y * (1.5 - 0.5 * x * y * y)
    return y


@functools.cache
def _make_fused(V, D, B):
    info = plsc.get_sparse_core_info()
    NC, NS, L = info.num_cores, info.num_subcores, info.num_lanes
    NW = NC * NS
    n_vreg = D // L
    b_per_w = B // NW
    C = 64
    NBUF = 4
    n_chunks = b_per_w // C
    assert B % (8 * NW) == 0 and D % L == 0 and C % L == 0 and n_chunks >= 4

    mesh = plsc.VectorSubcoreMesh(core_axis_name="c", subcore_axis_name="s")

    @functools.partial(
        pl.kernel,
        mesh=mesh,
        out_type=jax.ShapeDtypeStruct((B, D), jnp.float32),
        scratch_types=[
            pltpu.VMEM((b_per_w,), jnp.int32),
            pltpu.VMEM((NBUF * C, D), jnp.float32),
            pltpu.VMEM((D,), jnp.float32),
            pltpu.SemaphoreType.DMA((NBUF,)),
            pltpu.SemaphoreType.DMA((NBUF,)),
        ],
    )
    def k(ids_hbm, table_hbm, w_hbm, out_hbm, idx_v, buf, wv, gsem, ssem):
        wid = lax.axis_index("s") * NC + lax.axis_index("c")
        base = wid * b_per_w
        iota = jnp.arange(L, dtype=jnp.int32)
        masks = [jnp.asarray((iota & m) != 0) for m in (1, 2, 4, 8)]
        perms = [jnp.asarray(iota ^ m, dtype=jnp.int32) for m in (1, 2, 4, 8)]

        pltpu.sync_copy(w_hbm, wv)
        pltpu.sync_copy(ids_hbm.at[pl.ds(base, b_per_w)], idx_v)
        ws = [wv[pl.ds(j * L, L)] for j in range(n_vreg)]

        def bslice(slot):
            return buf.at[pl.ds(slot * C, C)]

        def gather(c, slot):
            pltpu.async_copy(
                table_hbm.at[idx_v.at[pl.ds(c * C, C)]], bslice(slot),
                gsem.at[slot],
            )

        def wait_gather(slot):
            pltpu.make_async_copy(
                table_hbm.at[pl.ds(0, C)], bslice(slot), gsem.at[slot]
            ).wait()

        def store(c, slot):
            pltpu.async_copy(
                bslice(slot), out_hbm.at[pl.ds(base + c * C, C)], ssem.at[slot]
            )

        def wait_store(slot):
            pltpu.make_async_copy(
                bslice(slot), out_hbm.at[pl.ds(base, C)], ssem.at[slot]
            ).wait()

        def combine(x, y, lvl):
            m = masks[lvl]
            return jnp.where(m, y, x) + _gather16(
                jnp.where(m, x, y), perms[lvl], L
            )

        def normalize_chunk(slot):
            row0 = slot * C

            @plsc.parallel_loop(0, C // L)
            def _(g):
                rbase = row0 + g * L
                partial = []
                for i in range(L):
                    a0 = jnp.zeros((L,), jnp.float32)
                    a1 = jnp.zeros((L,), jnp.float32)
                    a2 = jnp.zeros((L,), jnp.float32)
                    for j in range(0, n_vreg, 3):
                        v = buf[rbase + i, pl.ds(j * L, L)]
                        a0 = a0 + v * v
                        v = buf[rbase + i, pl.ds((j + 1) * L, L)]
                        a1 = a1 + v * v
                        v = buf[rbase + i, pl.ds((j + 2) * L, L)]
                        a2 = a2 + v * v
                    node, lvl = a0 + (a1 + a2), 0
                    while partial and partial[-1][0] == lvl:
                        node = combine(partial.pop()[1], node, lvl)
                        lvl += 1
                    partial.append((lvl, node))
                inv = _vrsqrt(partial[0][1] * (1.0 / D) + _EPS)
                for i in range(L):
                    sb = _gather16(inv, iota * 0 + i, L)
                    for j in range(n_vreg):
                        buf[rbase + i, pl.ds(j * L, L)] = (
                            buf[rbase + i, pl.ds(j * L, L)] * (ws[j] * sb)
                        )

        gather(0, 0)
        gather(1, 1)

        def chunk_body(c, carry):
            slot = lax.rem(c, NBUF)
            nslot = lax.rem(c + 2, NBUF)
            wait_gather(slot)
            normalize_chunk(slot)
            store(c, slot)

            @pl.when(c >= 2)
            def _():
                wait_store(nslot)

            @pl.when(c + 2 < n_chunks)
            def _():
                gather(c + 2, nslot)

            return carry

        lax.fori_loop(0, n_chunks, chunk_body, 0)
        for c in (n_chunks - 2, n_chunks - 1):
            wait_store(lax.rem(jnp.int32(c), NBUF))

    return k


def kernel(input_ids, tok_embeddings, norm_weight):
    Bt, S = input_ids.shape
    V, D = tok_embeddings.shape
    ids = input_ids.reshape(-1)
    out = _make_fused(V, D, Bt * S)(ids, tok_embeddings, norm_weight)
    return out.reshape(Bt, S, D)
